# Initial kernel scaffold; baseline (speedup 1.0000x reference)
import jax
import jax.numpy as jnp
from jax.experimental import pallas as pl


def kernel(frame_features, W1, b1, W2, b2, gamma_y, beta_y, gamma_l, beta_l, change_point):
    raise NotImplementedError("write your pallas kernel here")


# speedup vs baseline: 54.2366x; 54.2366x over previous
import functools

import jax
import jax.numpy as jnp
from jax import lax
from jax.experimental import pallas as pl
from jax.experimental.pallas import tpu as pltpu
from jax.experimental.pallas import tpu_sc as plsc

T = 16384
D = 1024
S = 512

A1, A2, A3 = 2048, 256, 32
OFF1 = 8
OFF2 = OFF1 + A1
OFF3 = OFF2 + A2
G_ROWS = OFF3 + A3

NC, NS, L = 2, 16, 16
NW = NC * NS
SEG_PER_W = S // NW
N0 = 16
NG = 64
NROWS = N0 + NG


def _build_body(f_ref, g_ref):
    x = f_ref[...]
    a1 = jnp.max(x.reshape(A1, 8, 128), axis=1)
    a2 = jnp.max(a1.reshape(A2, 8, 128), axis=1)
    a3 = jnp.max(a2.reshape(A3, 8, 128), axis=1)
    g_ref[0:OFF1, :] = jnp.full((OFF1, 128), -jnp.inf, jnp.float32)
    g_ref[OFF1:OFF2, :] = a1
    g_ref[OFF2:OFF3, :] = a2
    g_ref[OFF3:G_ROWS, :] = a3


def _build_table(frame_features):
    return pl.pallas_call(
        _build_body,
        grid=(D // 128,),
        in_specs=[pl.BlockSpec((T, 128), lambda i: (0, i))],
        out_specs=pl.BlockSpec((G_ROWS, 128), lambda i: (0, i)),
        out_shape=jax.ShapeDtypeStruct((G_ROWS, D), jnp.float32),
    )(frame_features)


def _next_level(s, e):
    return (s + 7) >> 3, ((e + 1) >> 3) - 1


def _segmax_body(f_hbm, g_hbm, s_hbm, e_hbm, out_hbm,
                 sv, ev, idx0, idxg, rows, outrow, sem0, sem1):
    wid = lax.axis_index("s") * NC + lax.axis_index("c")
    base = wid * SEG_PER_W

    pltpu.sync_copy(s_hbm.at[pl.ds(base, SEG_PER_W)], sv)
    pltpu.sync_copy(e_hbm.at[pl.ds(base, SEG_PER_W)], ev)
    s0 = sv[...]
    e0 = ev[...]
    lane = lax.iota(jnp.int32, L)

    for t in range(8):
        plsc.store_scatter(idx0, [lane * N0 + t], jnp.minimum(s0 + t, e0))
        plsc.store_scatter(idx0, [lane * N0 + (8 + t)], jnp.maximum(e0 - t, s0))

    sk, ek = s0, e0
    pos = 0
    for off in (OFF1, OFF2):
        sk, ek = _next_level(sk, ek)
        empty = sk > ek
        for t in range(8):
            li = jnp.where(empty, 0, jnp.minimum(sk + t, ek) + off)
            ri = jnp.where(empty, 0, jnp.maximum(ek - t, sk) + off)
            plsc.store_scatter(idxg, [lane * NG + pos + t], li)
            plsc.store_scatter(idxg, [lane * NG + pos + 8 + t], ri)
        pos += 16

    sk, ek = _next_level(sk, ek)
    empty = sk > ek
    for t in range(16):
        li = jnp.where(empty, 0, jnp.minimum(sk + t, ek) + OFF3)
        ri = jnp.where(empty, 0, jnp.maximum(ek - t, sk) + OFF3)
        plsc.store_scatter(idxg, [lane * NG + 32 + t], li)
        plsc.store_scatter(idxg, [lane * NG + 48 + t], ri)

    def seg_body(j, carry):
        c0 = pltpu.async_copy(
            f_hbm.at[idx0.at[pl.ds(j * N0, N0)]],
            rows.at[pl.ds(0, N0)], sem0)
        c1 = pltpu.async_copy(
            g_hbm.at[idxg.at[pl.ds(j * NG, NG)]],
            rows.at[pl.ds(N0, NG)], sem1)
        c0.wait()
        c1.wait()

        def v_body(v, c):
            def r_body(r, acc):
                return jnp.maximum(acc, rows[r, pl.ds(v * L, L)])
            acc = lax.fori_loop(
                0, NROWS, r_body,
                jnp.full((L,), -jnp.inf, jnp.float32), unroll=8)
            outrow[pl.ds(v * L, L)] = acc
            return c

        lax.fori_loop(0, D // L, v_body, 0)
        pltpu.sync_copy(outrow, out_hbm.at[base + j])
        return carry

    lax.fori_loop(0, SEG_PER_W, seg_body, 0)


def _segmax(frame_features, g, s_arr, e_arr):
    mesh = plsc.VectorSubcoreMesh(core_axis_name="c", subcore_axis_name="s")
    run = functools.partial(
        pl.kernel,
        out_type=jax.ShapeDtypeStruct((S, D), jnp.float32),
        mesh=mesh,
        compiler_params=pltpu.CompilerParams(needs_layout_passes=False),
        scratch_types=[
            pltpu.VMEM((SEG_PER_W,), jnp.int32),
            pltpu.VMEM((SEG_PER_W,), jnp.int32),
            pltpu.VMEM((SEG_PER_W * N0,), jnp.int32),
            pltpu.VMEM((SEG_PER_W * NG,), jnp.int32),
            pltpu.VMEM((NROWS, D), jnp.float32),
            pltpu.VMEM((D,), jnp.float32),
            pltpu.SemaphoreType.DMA,
            pltpu.SemaphoreType.DMA,
        ],
    )(_segmax_body)
    return run(frame_features, g, s_arr, e_arr)


def _ln(x, gamma, beta):
    mu = jnp.mean(x, axis=-1, keepdims=True)
    var = jnp.mean((x - mu) ** 2, axis=-1, keepdims=True)
    return (x - mu) / jnp.sqrt(var + 1e-6) * gamma + beta


def _mlp_body(x_ref, w1_ref, b1_ref, w2_ref, b2_ref,
              gy_ref, by_ref, gl_ref, bl_ref, out_ref):
    x = x_ref[...]
    y = _ln(x, gy_ref[...], by_ref[...])
    h = jnp.dot(y, w1_ref[...], preferred_element_type=jnp.float32)
    h = jax.nn.relu(h + b1_ref[...])
    h = _ln(h, gl_ref[...], bl_ref[...])
    logits = jnp.sum(h * w2_ref[...], axis=-1) + b2_ref[0, 0]
    out_ref[...] = jax.nn.sigmoid(logits)[None, :]


def _mlp(segs, W1, b1, W2, b2, gamma_y, beta_y, gamma_l, beta_l):
    full = lambda i: (0, 0)
    return pl.pallas_call(
        _mlp_body,
        grid=(S // 128,),
        in_specs=[
            pl.BlockSpec((128, D), lambda i: (i, 0)),
            pl.BlockSpec((D, D), full),
            pl.BlockSpec((1, D), full),
            pl.BlockSpec((1, D), full),
            pl.BlockSpec((1, 1), full),
            pl.BlockSpec((1, D), full),
            pl.BlockSpec((1, D), full),
            pl.BlockSpec((1, D), full),
            pl.BlockSpec((1, D), full),
        ],
        out_specs=pl.BlockSpec((1, 128), lambda i: (0, i)),
        out_shape=jax.ShapeDtypeStruct((1, S), jnp.float32),
    )(segs, W1, b1.reshape(1, D), W2.reshape(1, D), b2.reshape(1, 1),
      gamma_y.reshape(1, D), beta_y.reshape(1, D),
      gamma_l.reshape(1, D), beta_l.reshape(1, D))


def kernel(frame_features, W1, b1, W2, b2,
           gamma_y, beta_y, gamma_l, beta_l, change_point):
    cp = change_point.astype(jnp.int32)
    s_arr = cp[:, 0]
    e_arr = cp[:, 1]
    g = _build_table(frame_features)
    segs = _segmax(frame_features, g, s_arr, e_arr)
    return _mlp(segs, W1, b1, W2, b2, gamma_y, beta_y, gamma_l, beta_l)
